# baseline (device time: 27515 ns/iter reference)
import os

import jax
import jax.numpy as jnp
from jax import lax
from jax.experimental import pallas as pl
from jax.experimental.pallas import tpu as pltpu

N_DEV = 32
B, SQ, D = 2, 128, 512
ROWS = B * SQ
CH = ROWS // N_DEV
CPB = N_DEV // B
H_LOC = 8
DH = 64
KV_PER_SHARD = 2

_PROBE = os.environ.get("KERNEL_PROBE", "")

_OFFSETS = [18, 14, 22, 10, 19, 13, 21, 11, 20, 12, 26, 6, 27, 17, 15, 5,
            30, 2, 29, 3, 23, 9, 28, 4, 25, 7, 16, 24, 8, 31, 1]


def kernel(x, Wq, Wo, K_ext, V_ext):
    idx = lax.axis_index("i")
    K_loc = lax.dynamic_slice_in_dim(K_ext, idx * KV_PER_SHARD, KV_PER_SHARD, axis=2)
    V_loc = lax.dynamic_slice_in_dim(V_ext, idx * KV_PER_SHARD, KV_PER_SHARD, axis=2)
    KT_loc = jnp.transpose(K_loc, (0, 2, 3, 1))
    V_loc = jnp.transpose(V_loc, (0, 2, 1, 3))

    def body(x_ref, wq_ref, wo_ref, kt_ref, v_ref, out_ref,
             sbuf, gbuf, obuf, ssem1, rsem1, ssem2, rsem2):
        my = lax.axis_index("i")

        def when_not_me(j, fn):
            pl.when(my != j)(fn)

        if _PROBE != "compute":
            bsem = pltpu.get_barrier_semaphore()
            for d in range(N_DEV):
                def _sig(d=d):
                    pl.semaphore_signal(
                        bsem, inc=1,
                        device_id=(d,),
                        device_id_type=pl.DeviceIdType.MESH,
                    )
                when_not_me(d, _sig)

        targets = [lax.rem(my + off, N_DEV) for off in _OFFSETS]

        x16 = x_ref[...].astype(jnp.bfloat16)
        wq16 = wq_ref[...].astype(jnp.bfloat16)
        wo16 = wo_ref[...].astype(jnp.bfloat16)
        q_all = jnp.dot(
            x16, wq16, preferred_element_type=jnp.float32
        )
        GQ = H_LOC // KV_PER_SHARD
        zk = jnp.zeros((DH, SQ), jnp.bfloat16)
        zv = jnp.zeros((SQ, DH), jnp.bfloat16)
        for b in range(B):
            qb = q_all[b * SQ:(b + 1) * SQ].astype(jnp.bfloat16)
            kv_outs = []
            for g in range(KV_PER_SHARD):
                q4 = qb[:, g * GQ * DH:(g + 1) * GQ * DH]
                kt = kt_ref[b, g].astype(jnp.bfloat16)
                v = v_ref[b, g].astype(jnp.bfloat16)
                k4t = jnp.concatenate(
                    [
                        jnp.concatenate(
                            [kt if c == r else zk for c in range(GQ)], axis=1
                        )
                        for r in range(GQ)
                    ],
                    axis=0,
                )
                v4 = jnp.concatenate(
                    [
                        jnp.concatenate(
                            [v if c == r else zv for c in range(GQ)], axis=1
                        )
                        for r in range(GQ)
                    ],
                    axis=0,
                )
                s4 = jnp.dot(
                    q4, k4t, preferred_element_type=jnp.float32
                ) * 0.125
                ps = []
                recips = []
                for hh in range(GQ):
                    p = jnp.exp(s4[:, hh * SQ:(hh + 1) * SQ])
                    l = jnp.sum(p, axis=1, keepdims=True)
                    ps.append(p.astype(jnp.bfloat16))
                    recips.append(
                        jnp.broadcast_to(jnp.reciprocal(l), (SQ, DH))
                    )
                p4 = jnp.concatenate(ps, axis=1)
                o4 = jnp.dot(
                    p4, v4, preferred_element_type=jnp.float32
                )
                kv_outs.append(o4 * jnp.concatenate(recips, axis=1))
            attn = jnp.concatenate(kv_outs, axis=1)
            partial = jnp.dot(
                attn.astype(jnp.bfloat16), wo16,
                preferred_element_type=jnp.float32,
            )
            sbuf[b * CPB:(b + 1) * CPB] = partial.astype(jnp.bfloat16).reshape(
                CPB, CH, D
            )

        if _PROBE == "compute":
            out_ref[...] = jnp.zeros((ROWS, D), jnp.float32)
            return

        pl.semaphore_wait(bsem, N_DEV - 1)

        for oi, t in enumerate(targets):
            rdma = pltpu.make_async_remote_copy(
                src_ref=sbuf.at[t],
                dst_ref=gbuf.at[my],
                send_sem=ssem1.at[oi],
                recv_sem=rsem1.at[my],
                device_id=(t,),
                device_id_type=pl.DeviceIdType.MESH,
            )
            rdma.start()

        gbuf[my] = sbuf[my]

        for t in targets:
            recv = pltpu.make_async_remote_copy(
                src_ref=gbuf.at[t],
                dst_ref=gbuf.at[t],
                send_sem=ssem1.at[0],
                recv_sem=rsem1.at[t],
                device_id=(t,),
                device_id_type=pl.DeviceIdType.MESH,
            )
            recv.wait_recv()

        reduced = jnp.sum(gbuf[...].astype(jnp.float32), axis=0)
        obuf[my] = reduced.astype(jnp.bfloat16)

        if _PROBE == "phase1":
            for oi in range(N_DEV - 1):
                snd = pltpu.make_async_remote_copy(
                    src_ref=sbuf.at[oi],
                    dst_ref=gbuf.at[oi],
                    send_sem=ssem1.at[oi],
                    recv_sem=rsem1.at[oi],
                    device_id=(oi,),
                    device_id_type=pl.DeviceIdType.MESH,
                )
                snd.wait_send()
            out_ref[...] = jnp.zeros((ROWS, D), jnp.float32)
            return

        for oi, t in enumerate(targets):
            rdma = pltpu.make_async_remote_copy(
                src_ref=obuf.at[my],
                dst_ref=obuf.at[my],
                send_sem=ssem2.at[oi],
                recv_sem=rsem2.at[my],
                device_id=(t,),
                device_id_type=pl.DeviceIdType.MESH,
            )
            rdma.start()

        for t in targets:
            recv = pltpu.make_async_remote_copy(
                src_ref=obuf.at[t],
                dst_ref=obuf.at[t],
                send_sem=ssem2.at[0],
                recv_sem=rsem2.at[t],
                device_id=(t,),
                device_id_type=pl.DeviceIdType.MESH,
            )
            recv.wait_recv()

        out_ref[...] = obuf[...].astype(jnp.float32).reshape(ROWS, D)

        for oi in range(N_DEV - 1):
            for ssem, src in ((ssem1, sbuf), (ssem2, obuf)):
                snd = pltpu.make_async_remote_copy(
                    src_ref=src.at[oi],
                    dst_ref=src.at[oi],
                    send_sem=ssem.at[oi],
                    recv_sem=rsem1.at[oi],
                    device_id=(oi,),
                    device_id_type=pl.DeviceIdType.MESH,
                )
                snd.wait_send()

    out2 = pl.pallas_call(
        body,
        out_shape=jax.ShapeDtypeStruct((ROWS, D), jnp.float32),
        in_specs=[pl.BlockSpec(memory_space=pltpu.VMEM)] * 5,
        out_specs=pl.BlockSpec(memory_space=pltpu.VMEM),
        scratch_shapes=[
            pltpu.VMEM((N_DEV, CH, D), jnp.bfloat16),
            pltpu.VMEM((N_DEV, CH, D), jnp.bfloat16),
            pltpu.VMEM((N_DEV, CH, D), jnp.bfloat16),
            pltpu.SemaphoreType.DMA((N_DEV,)),
            pltpu.SemaphoreType.DMA((N_DEV,)),
            pltpu.SemaphoreType.DMA((N_DEV,)),
            pltpu.SemaphoreType.DMA((N_DEV,)),
        ],
        compiler_params=pltpu.CompilerParams(
            collective_id=None if _PROBE == "compute" else 0
        ),
    )(x.reshape(ROWS, D), Wq, Wo, KT_loc, V_loc)
    return out2.reshape(B, SQ, D)


# device time: 26479 ns/iter; 1.0391x vs baseline; 1.0391x over previous
import os

import jax
import jax.numpy as jnp
from jax import lax
from jax.experimental import pallas as pl
from jax.experimental.pallas import tpu as pltpu

N_DEV = 32
B, SQ, D = 2, 128, 512
ROWS = B * SQ
CH = ROWS // N_DEV
CPB = N_DEV // B
H_LOC = 8
DH = 64
KV_PER_SHARD = 2

_PROBE = os.environ.get("KERNEL_PROBE", "")


def kernel(x, Wq, Wo, K_ext, V_ext):
    idx = lax.axis_index("i")
    K_loc = lax.dynamic_slice_in_dim(K_ext, idx * KV_PER_SHARD, KV_PER_SHARD, axis=2)
    V_loc = lax.dynamic_slice_in_dim(V_ext, idx * KV_PER_SHARD, KV_PER_SHARD, axis=2)
    KT_loc = jnp.transpose(K_loc, (0, 2, 3, 1))
    V_loc = jnp.transpose(V_loc, (0, 2, 1, 3))

    def body(x_ref, wq_ref, wo_ref, kt_ref, v_ref, out_ref,
             sbuf, gbuf, obuf, ssem1, rsem1, ssem2, rsem2):
        my = lax.axis_index("i")

        def when_not_me(j, fn):
            pl.when(my != j)(fn)

        if _PROBE != "compute":
            bsem = pltpu.get_barrier_semaphore()
            for d in range(N_DEV):
                def _sig(d=d):
                    pl.semaphore_signal(
                        bsem, inc=1,
                        device_id=(d,),
                        device_id_type=pl.DeviceIdType.MESH,
                    )
                when_not_me(d, _sig)

        def send_chunk(j):
            if _PROBE == "compute":
                return
            rdma = pltpu.make_async_remote_copy(
                src_ref=sbuf.at[j],
                dst_ref=gbuf.at[my],
                send_sem=ssem1.at[j],
                recv_sem=rsem1.at[my],
                device_id=(j,),
                device_id_type=pl.DeviceIdType.MESH,
            )
            when_not_me(j, rdma.start)

        x16 = x_ref[...].astype(jnp.bfloat16)
        wq16 = wq_ref[...].astype(jnp.bfloat16)
        wo16 = wo_ref[...].astype(jnp.bfloat16)
        q_all = jnp.dot(
            x16, wq16, preferred_element_type=jnp.float32
        )
        GQ = H_LOC // KV_PER_SHARD
        zk = jnp.zeros((DH, SQ), jnp.bfloat16)
        zv = jnp.zeros((SQ, DH), jnp.bfloat16)
        for b in range(B):
            qb = q_all[b * SQ:(b + 1) * SQ].astype(jnp.bfloat16)
            kv_outs = []
            for g in range(KV_PER_SHARD):
                q4 = qb[:, g * GQ * DH:(g + 1) * GQ * DH]
                kt = kt_ref[b, g].astype(jnp.bfloat16)
                v = v_ref[b, g].astype(jnp.bfloat16)
                k4t = jnp.concatenate(
                    [
                        jnp.concatenate(
                            [kt if c == r else zk for c in range(GQ)], axis=1
                        )
                        for r in range(GQ)
                    ],
                    axis=0,
                )
                v4 = jnp.concatenate(
                    [
                        jnp.concatenate(
                            [v if c == r else zv for c in range(GQ)], axis=1
                        )
                        for r in range(GQ)
                    ],
                    axis=0,
                )
                s4 = jnp.dot(
                    q4, k4t, preferred_element_type=jnp.float32
                ) * 0.125
                ps = []
                recips = []
                for hh in range(GQ):
                    p = jnp.exp(s4[:, hh * SQ:(hh + 1) * SQ])
                    l = jnp.sum(p, axis=1, keepdims=True)
                    ps.append(p.astype(jnp.bfloat16))
                    recips.append(
                        jnp.broadcast_to(jnp.reciprocal(l), (SQ, DH))
                    )
                p4 = jnp.concatenate(ps, axis=1)
                o4 = jnp.dot(
                    p4, v4, preferred_element_type=jnp.float32
                )
                kv_outs.append(o4 * jnp.concatenate(recips, axis=1))
            attn = jnp.concatenate(kv_outs, axis=1)
            partial = jnp.dot(
                attn.astype(jnp.bfloat16), wo16,
                preferred_element_type=jnp.float32,
            )
            sbuf[b * CPB:(b + 1) * CPB] = partial.astype(jnp.bfloat16).reshape(
                CPB, CH, D
            )
            if b == 0 and _PROBE != "compute":
                pl.semaphore_wait(bsem, N_DEV - 1)
            for j in range(b * CPB, (b + 1) * CPB):
                send_chunk(j)

        if _PROBE == "compute":
            out_ref[...] = jnp.zeros((ROWS, D), jnp.float32)
            return

        gbuf[my] = sbuf[my]

        for k in range(N_DEV):
            recv = pltpu.make_async_remote_copy(
                src_ref=gbuf.at[k],
                dst_ref=gbuf.at[k],
                send_sem=ssem1.at[k],
                recv_sem=rsem1.at[k],
                device_id=(k,),
                device_id_type=pl.DeviceIdType.MESH,
            )
            when_not_me(k, recv.wait_recv)

        reduced = jnp.sum(gbuf[...].astype(jnp.float32), axis=0)
        obuf[my] = reduced.astype(jnp.bfloat16)

        if _PROBE == "phase1":
            for j in range(N_DEV):
                snd = pltpu.make_async_remote_copy(
                    src_ref=sbuf.at[j],
                    dst_ref=gbuf.at[j],
                    send_sem=ssem1.at[j],
                    recv_sem=rsem1.at[j],
                    device_id=(j,),
                    device_id_type=pl.DeviceIdType.MESH,
                )
                when_not_me(j, snd.wait_send)
            out_ref[...] = jnp.zeros((ROWS, D), jnp.float32)
            return

        for d in range(N_DEV):
            rdma = pltpu.make_async_remote_copy(
                src_ref=obuf.at[my],
                dst_ref=obuf.at[my],
                send_sem=ssem2.at[d],
                recv_sem=rsem2.at[my],
                device_id=(d,),
                device_id_type=pl.DeviceIdType.MESH,
            )
            when_not_me(d, rdma.start)

        for k in range(N_DEV):
            recv = pltpu.make_async_remote_copy(
                src_ref=obuf.at[k],
                dst_ref=obuf.at[k],
                send_sem=ssem2.at[k],
                recv_sem=rsem2.at[k],
                device_id=(k,),
                device_id_type=pl.DeviceIdType.MESH,
            )
            when_not_me(k, recv.wait_recv)

        out_ref[...] = obuf[...].astype(jnp.float32).reshape(ROWS, D)

        for j in range(N_DEV):
            for ssem, src in ((ssem1, sbuf), (ssem2, obuf)):
                snd = pltpu.make_async_remote_copy(
                    src_ref=src.at[j],
                    dst_ref=src.at[j],
                    send_sem=ssem.at[j],
                    recv_sem=rsem1.at[j],
                    device_id=(j,),
                    device_id_type=pl.DeviceIdType.MESH,
                )
                when_not_me(j, snd.wait_send)

    out2 = pl.pallas_call(
        body,
        out_shape=jax.ShapeDtypeStruct((ROWS, D), jnp.float32),
        in_specs=[pl.BlockSpec(memory_space=pltpu.VMEM)] * 5,
        out_specs=pl.BlockSpec(memory_space=pltpu.VMEM),
        scratch_shapes=[
            pltpu.VMEM((N_DEV, CH, D), jnp.bfloat16),
            pltpu.VMEM((N_DEV, CH, D), jnp.bfloat16),
            pltpu.VMEM((N_DEV, CH, D), jnp.bfloat16),
            pltpu.SemaphoreType.DMA((N_DEV,)),
            pltpu.SemaphoreType.DMA((N_DEV,)),
            pltpu.SemaphoreType.DMA((N_DEV,)),
            pltpu.SemaphoreType.DMA((N_DEV,)),
        ],
        compiler_params=pltpu.CompilerParams(
            collective_id=None if _PROBE == "compute" else 0
        ),
    )(x.reshape(ROWS, D), Wq, Wo, KT_loc, V_loc)
    return out2.reshape(B, SQ, D)
